# baseline (speedup 1.0000x reference)
import functools

import jax
import jax.numpy as jnp
from jax import lax
from jax.experimental import pallas as pl
from jax.experimental.pallas import tpu as pltpu
from jax.experimental.pallas import tpu_sc as plsc

NC, NS, L = 2, 16, 16
NW = NC * NS
N, E, D, H = 10000, 320000, 128, 128
NP = 10240
CH = 128
NCHUNK = 2504
E2 = NCHUNK * CH
G8 = 8
NG = NCHUNK // G8
ROWS_PER_TILE = NP // NS


def _mesh():
    return plsc.VectorSubcoreMesh(
        core_axis_name="c", subcore_axis_name="s",
        num_cores=NC, num_subcores=NS)


def _rsqrt_newton(d):
    i = lax.bitcast_convert_type(d, jnp.int32)
    y = lax.bitcast_convert_type(jnp.int32(0x5F3759DF) - (i >> 1), jnp.float32)
    for _ in range(3):
        y = y * (1.5 - 0.5 * d * y * y)
    return y


@functools.partial(
    pl.kernel,
    out_type=jax.ShapeDtypeStruct((NC, NP), jnp.float32),
    mesh=_mesh(),
    scratch_types=[
        pltpu.VMEM_SHARED((NP,), jnp.float32),
        pltpu.VMEM((1, G8, CH), jnp.int32),
        pltpu.VMEM((1, G8, CH), jnp.float32),
        pltpu.VMEM((ROWS_PER_TILE,), jnp.float32),
        pltpu.SemaphoreType.DMA,
    ],
)
def _k1_deg(dst3_hbm, w3_hbm, degp_hbm, deg_sp, didx, wst, vb, sem):
    c = lax.axis_index("c")
    s = lax.axis_index("s")
    w = s * NC + c

    def zstep(k, _):
        vb[pl.ds(k * L, L)] = jnp.zeros((L,), jnp.float32)
        return 0
    lax.fori_loop(0, ROWS_PER_TILE // L, zstep, 0)
    pltpu.sync_copy(vb, deg_sp.at[pl.ds(s * ROWS_PER_TILE, ROWS_PER_TILE)])
    plsc.subcore_barrier()

    ngroups = 9 + jnp.where(w < NG - 9 * NW, 1, 0)

    def group(i, _):
        g = w + NW * i
        pltpu.sync_copy(dst3_hbm.at[pl.ds(g, 1), :, :], didx)
        pltpu.sync_copy(w3_hbm.at[pl.ds(g, 1), :, :], wst)
        ds_ = [pltpu.async_copy(wst.at[0, j], deg_sp.at[didx.at[0, j]], sem,
                                add=True)
               for j in range(G8)]
        for d in ds_:
            d.wait()
        return 0
    lax.fori_loop(0, ngroups, group, 0)
    plsc.subcore_barrier()

    pltpu.sync_copy(deg_sp.at[pl.ds(s * ROWS_PER_TILE, ROWS_PER_TILE)], vb)
    pltpu.sync_copy(vb, degp_hbm.at[c].at[pl.ds(s * ROWS_PER_TILE,
                                                ROWS_PER_TILE)])


def _k1b_body(degp_ref, dinv_ref):
    dinv_ref[...] = lax.rsqrt(
        jnp.maximum(degp_ref[0] + degp_ref[1], 1e-6))[None, :]


def _k1b_call(degp):
    nblk = 10
    cb = NP // nblk
    return pl.pallas_call(
        _k1b_body,
        grid=(nblk,),
        in_specs=[pl.BlockSpec((NC, cb), lambda i: (0, i))],
        out_specs=pl.BlockSpec((1, cb), lambda i: (0, i)),
        out_shape=jax.ShapeDtypeStruct((1, NP), jnp.float32),
    )(degp)


def _k1c_body(x_ref, dcol_ref, xs_ref):
    xs_ref[...] = x_ref[...] * dcol_ref[...]


def _k1c_call(x_pad, dcol):
    nblk = 10
    rb = NP // nblk
    return pl.pallas_call(
        _k1c_body,
        grid=(nblk,),
        in_specs=[
            pl.BlockSpec((rb, D), lambda i: (i, 0)),
            pl.BlockSpec((rb, 1), lambda i: (i, 0)),
        ],
        out_specs=pl.BlockSpec((rb, D), lambda i: (i, 0)),
        out_shape=jax.ShapeDtypeStruct((NP, D), jnp.float32),
    )(x_pad, dcol)


PER_SC = NCHUNK // NC


@functools.partial(
    pl.kernel,
    out_type=jax.ShapeDtypeStruct((NC, NP, D), jnp.float32),
    mesh=_mesh(),
    scratch_types=[
        pltpu.VMEM_SHARED((NP, D), jnp.float32),
        pltpu.VMEM((3, CH), jnp.int32),
        pltpu.VMEM((3, CH), jnp.int32),
        pltpu.VMEM((3, CH), jnp.float32),
        pltpu.VMEM((2, CH, D), jnp.float32),
        pltpu.SemaphoreType.DMA((3,)),
        pltpu.SemaphoreType.DMA((3,)),
        pltpu.SemaphoreType.DMA((2,)),
    ],
)
def _k2_agg(src_hbm, dst_hbm, w_hbm, xs_hbm, aggp_hbm,
            agg_sp, sidx, didx, wsl, xrows, semi, semg, sems):
    c = lax.axis_index("c")
    s = lax.axis_index("s")

    for r in range(64):
        for k in range(D // L):
            xrows[0, r, pl.ds(k * L, L)] = jnp.zeros((L,), jnp.float32)

    def zstep(j, _):
        pltpu.sync_copy(xrows.at[0].at[pl.ds(0, 64), :],
                        agg_sp.at[pl.ds(s * ROWS_PER_TILE + j * 64, 64), :])
        return 0
    lax.fori_loop(0, ROWS_PER_TILE // 64, zstep, 0)
    plsc.subcore_barrier()

    nch = 78 + jnp.where(s < PER_SC - 78 * NS, 1, 0)

    def chunk_of(i):
        return c * PER_SC + s + NS * i

    def start_idx(i):
        sl = lax.rem(i, 3)
        ck = chunk_of(i)
        pltpu.async_copy(src_hbm.at[ck], sidx.at[sl], semi.at[sl])
        pltpu.async_copy(dst_hbm.at[ck], didx.at[sl], semi.at[sl])
        pltpu.async_copy(w_hbm.at[ck], wsl.at[sl], semi.at[sl])

    def wait_idx(i):
        sl = lax.rem(i, 3)
        ck = chunk_of(i)
        pltpu.make_async_copy(src_hbm.at[ck], sidx.at[sl], semi.at[sl]).wait()
        pltpu.make_async_copy(dst_hbm.at[ck], didx.at[sl], semi.at[sl]).wait()
        pltpu.make_async_copy(w_hbm.at[ck], wsl.at[sl], semi.at[sl]).wait()

    def start_gather(i):
        sl = lax.rem(i, 3)
        x2 = lax.rem(i, 2)
        pltpu.async_copy(xs_hbm.at[sidx.at[sl]], xrows.at[x2], semg.at[sl])

    def wait_gather(i):
        sl = lax.rem(i, 3)
        x2 = lax.rem(i, 2)
        pltpu.make_async_copy(xs_hbm.at[sidx.at[sl]], xrows.at[x2],
                              semg.at[sl]).wait()


    start_idx(0)

    @pl.when(nch > 1)
    def _():
        start_idx(1)
    wait_idx(0)
    start_gather(0)

    def wait_scatter(i):
        sl = lax.rem(i, 3)
        x2 = lax.rem(i, 2)
        pltpu.make_async_copy(xrows.at[x2], agg_sp.at[didx.at[sl]],
                              sems.at[x2]).wait()

    def step(i, _):
        sl = lax.rem(i, 3)

        @pl.when(i + 1 < nch)
        def _():
            wait_idx(i + 1)

        @pl.when(i >= 1)
        def _():
            wait_scatter(i - 1)

        @pl.when(i + 1 < nch)
        def _():
            start_gather(i + 1)

        @pl.when(i + 2 < nch)
        def _():
            start_idx(i + 2)

        wait_gather(i)
        x2 = lax.rem(i, 2)

        def scale(g, _):
            nv = wsl[sl, pl.ds(pl.multiple_of(g * L, L), L)]
            for j in range(L):
                e = g * L + j
                sc = nv[j]
                for k in range(D // L):
                    xrows[x2, e, pl.ds(k * L, L)] = (
                        xrows[x2, e, pl.ds(k * L, L)] * sc)
            return 0
        lax.fori_loop(0, CH // L, scale, 0)
        pltpu.async_copy(xrows.at[x2], agg_sp.at[didx.at[sl]], sems.at[x2],
                         add=True)
        return 0
    lax.fori_loop(0, nch, step, 0)
    wait_scatter(nch - 1)
    plsc.subcore_barrier()

    def ostep(j, _):
        r = s * ROWS_PER_TILE + j * 64
        pltpu.sync_copy(agg_sp.at[pl.ds(r, 64), :],
                        xrows.at[0].at[pl.ds(0, 64), :])
        pltpu.sync_copy(xrows.at[0].at[pl.ds(0, 64), :],
                        aggp_hbm.at[c].at[pl.ds(r, 64), :])
        return 0
    lax.fori_loop(0, ROWS_PER_TILE // 64, ostep, 0)


def _k3_body(aggp_ref, dcol_ref, wg_ref, bg_ref, w1a_ref, w1b_ref, b1_ref,
             a_ref, b_ref):
    agg = (aggp_ref[0] + aggp_ref[1]) * dcol_ref[...]
    emb = jnp.maximum(
        jnp.dot(agg, wg_ref[...], preferred_element_type=jnp.float32)
        + bg_ref[...], 0.0)
    a_ref[...] = (jnp.dot(emb, w1a_ref[...], preferred_element_type=jnp.float32)
                  + b1_ref[...])
    b_ref[...] = jnp.dot(emb, w1b_ref[...], preferred_element_type=jnp.float32)


def _k3_call(aggp, dcol, wg, bg, w1a, w1b, b1):
    nblk = 10
    rb = NP // nblk
    return pl.pallas_call(
        _k3_body,
        grid=(nblk,),
        in_specs=[
            pl.BlockSpec((NC, rb, D), lambda i: (0, i, 0)),
            pl.BlockSpec((rb, 1), lambda i: (i, 0)),
            pl.BlockSpec((D, H), lambda i: (0, 0)),
            pl.BlockSpec((1, H), lambda i: (0, 0)),
            pl.BlockSpec((H, H), lambda i: (0, 0)),
            pl.BlockSpec((H, H), lambda i: (0, 0)),
            pl.BlockSpec((1, H), lambda i: (0, 0)),
        ],
        out_specs=[
            pl.BlockSpec((rb, H), lambda i: (i, 0)),
            pl.BlockSpec((rb, H), lambda i: (i, 0)),
        ],
        out_shape=[
            jax.ShapeDtypeStruct((NP, H), jnp.float32),
            jax.ShapeDtypeStruct((NP, H), jnp.float32),
        ],
    )(aggp, dcol, wg, bg, w1a, w1b, b1)


@functools.partial(
    pl.kernel,
    out_type=jax.ShapeDtypeStruct((E2,), jnp.float32),
    mesh=_mesh(),
    scratch_types=[
        pltpu.VMEM((3, CH), jnp.int32),
        pltpu.VMEM((3, CH), jnp.int32),
        pltpu.VMEM((3, CH, H), jnp.float32),
        pltpu.VMEM((3, CH, H), jnp.float32),
        pltpu.VMEM((144,), jnp.float32),
        pltpu.VMEM((3, CH), jnp.float32),
        pltpu.SemaphoreType.DMA((3,)),
        pltpu.SemaphoreType.DMA((3,)),
        pltpu.SemaphoreType.DMA((3,)),
    ],
)
def _k4_decode(src_hbm, dst_hbm, a_hbm, b_hbm, par_hbm, out_hbm,
               sidx, didx, arows, brows, parv, obuf, semi, semg, semo):
    c = lax.axis_index("c")
    s = lax.axis_index("s")
    w = s * NC + c

    pltpu.sync_copy(par_hbm, parv)
    w2 = [parv[pl.ds(k * L, L)] for k in range(H // L)]
    b2 = parv[pl.ds(H, L)][0]
    i16 = lax.iota(jnp.int32, L)

    nch = 78 + jnp.where(w < NCHUNK - 78 * NW, 1, 0)

    def chunk_of(i):
        return w + NW * i

    def start_idx(i):
        sl = lax.rem(i, 3)
        ck = chunk_of(i)
        pltpu.async_copy(src_hbm.at[ck], sidx.at[sl], semi.at[sl])
        pltpu.async_copy(dst_hbm.at[ck], didx.at[sl], semi.at[sl])

    def wait_idx(i):
        sl = lax.rem(i, 3)
        ck = chunk_of(i)
        pltpu.make_async_copy(src_hbm.at[ck], sidx.at[sl], semi.at[sl]).wait()
        pltpu.make_async_copy(dst_hbm.at[ck], didx.at[sl], semi.at[sl]).wait()

    def start_gather(i):
        sl = lax.rem(i, 3)
        pltpu.async_copy(a_hbm.at[sidx.at[sl]], arows.at[sl], semg.at[sl])
        pltpu.async_copy(b_hbm.at[didx.at[sl]], brows.at[sl], semg.at[sl])

    def wait_gather(i):
        sl = lax.rem(i, 3)
        pltpu.make_async_copy(a_hbm.at[sidx.at[sl]], arows.at[sl],
                              semg.at[sl]).wait()
        pltpu.make_async_copy(b_hbm.at[didx.at[sl]], brows.at[sl],
                              semg.at[sl]).wait()

    def wait_out(i):
        sl = lax.rem(i, 3)
        base = chunk_of(i) * CH
        pltpu.make_async_copy(obuf.at[sl], out_hbm.at[pl.ds(base, CH)],
                              semo.at[sl]).wait()

    start_idx(0)

    @pl.when(nch > 1)
    def _():
        start_idx(1)
    wait_idx(0)
    start_gather(0)

    def step(i, _):
        sl = lax.rem(i, 3)

        @pl.when(i + 1 < nch)
        def _():
            wait_idx(i + 1)
            start_gather(i + 1)

        @pl.when(i + 2 < nch)
        def _():
            start_idx(i + 2)

        wait_gather(i)

        @pl.when(i >= 3)
        def _():
            wait_out(i - 3)

        def group(g, _):
            z = jnp.zeros((L,), jnp.float32)
            for j in range(L):
                e = g * L + j
                acc = jnp.zeros((L,), jnp.float32)
                for k in range(H // L):
                    t = jnp.maximum(
                        arows[sl, e, pl.ds(k * L, L)]
                        + brows[sl, e, pl.ds(k * L, L)], 0.0)
                    acc = acc + t * w2[k]
                sval = acc[0]
                for m in range(1, L):
                    sval = sval + acc[m]
                z = jnp.where(i16 == j, sval, z)
            sig = 1.0 / (1.0 + jnp.exp(-(z + b2)))
            obuf[sl, pl.ds(pl.multiple_of(g * L, L), L)] = sig
            return 0
        lax.fori_loop(0, CH // L, group, 0)
        base = chunk_of(i) * CH
        pltpu.async_copy(obuf.at[sl], out_hbm.at[pl.ds(base, CH)],
                         semo.at[sl])
        return 0
    lax.fori_loop(0, nch, step, 0)

    wait_out(nch - 3)
    wait_out(nch - 2)
    wait_out(nch - 1)


def kernel(x, edge_index, edge_weight, W_gcn, b_gcn, W1, b1, W2, b2):
    pad = E2 - E
    src = jnp.pad(edge_index[0], (0, pad)).reshape(NCHUNK, CH)
    dst = jnp.pad(edge_index[1], (0, pad)).reshape(NCHUNK, CH)
    wgt = jnp.pad(edge_weight, (0, pad)).reshape(NCHUNK, CH)
    degp = _k1_deg(dst.reshape(NG, G8, CH), wgt.reshape(NG, G8, CH))
    dcol = _k1b_call(degp).reshape(NP, 1)
    x_pad = jnp.pad(x, ((0, NP - N), (0, 0)))
    xs = _k1c_call(x_pad, dcol)
    aggp = _k2_agg(src, dst, wgt, xs)
    a_mat, b_mat = _k3_call(
        aggp, dcol, W_gcn, b_gcn.reshape(1, H), W1[:H], W1[H:],
        b1.reshape(1, H))
    params = jnp.concatenate(
        [W2[:, 0], b2, jnp.zeros((15,), jnp.float32)])
    out = _k4_decode(src, dst, a_mat, b_mat, params)
    return out[:E]


# speedup vs baseline: 1.0262x; 1.0262x over previous
import functools

import jax
import jax.numpy as jnp
from jax import lax
from jax.experimental import pallas as pl
from jax.experimental.pallas import tpu as pltpu
from jax.experimental.pallas import tpu_sc as plsc

NC, NS, L = 2, 16, 16
NW = NC * NS
N, E, D, H = 10000, 320000, 128, 128
NP = 10240
CH = 128
NCHUNK = 2504
E2 = NCHUNK * CH
G8 = 8
NG = NCHUNK // G8
ROWS_PER_TILE = NP // NS


def _mesh():
    return plsc.VectorSubcoreMesh(
        core_axis_name="c", subcore_axis_name="s",
        num_cores=NC, num_subcores=NS)


def _rsqrt_newton(d):
    i = lax.bitcast_convert_type(d, jnp.int32)
    y = lax.bitcast_convert_type(jnp.int32(0x5F3759DF) - (i >> 1), jnp.float32)
    for _ in range(3):
        y = y * (1.5 - 0.5 * d * y * y)
    return y


@functools.partial(
    pl.kernel,
    out_type=jax.ShapeDtypeStruct((NC, NP), jnp.float32),
    mesh=_mesh(),
    scratch_types=[
        pltpu.VMEM_SHARED((NP,), jnp.float32),
        pltpu.VMEM((1, G8, CH), jnp.int32),
        pltpu.VMEM((1, G8, CH), jnp.float32),
        pltpu.VMEM((ROWS_PER_TILE,), jnp.float32),
        pltpu.SemaphoreType.DMA,
    ],
)
def _k1_deg(dst3_hbm, w3_hbm, degp_hbm, deg_sp, didx, wst, vb, sem):
    c = lax.axis_index("c")
    s = lax.axis_index("s")
    w = s * NC + c

    def zstep(k, _):
        vb[pl.ds(k * L, L)] = jnp.zeros((L,), jnp.float32)
        return 0
    lax.fori_loop(0, ROWS_PER_TILE // L, zstep, 0)
    pltpu.sync_copy(vb, deg_sp.at[pl.ds(s * ROWS_PER_TILE, ROWS_PER_TILE)])
    plsc.subcore_barrier()

    ngroups = 9 + jnp.where(w < NG - 9 * NW, 1, 0)

    def group(i, _):
        g = w + NW * i
        pltpu.sync_copy(dst3_hbm.at[pl.ds(g, 1), :, :], didx)
        pltpu.sync_copy(w3_hbm.at[pl.ds(g, 1), :, :], wst)
        ds_ = [pltpu.async_copy(wst.at[0, j], deg_sp.at[didx.at[0, j]], sem,
                                add=True)
               for j in range(G8)]
        for d in ds_:
            d.wait()
        return 0
    lax.fori_loop(0, ngroups, group, 0)
    plsc.subcore_barrier()

    pltpu.sync_copy(deg_sp.at[pl.ds(s * ROWS_PER_TILE, ROWS_PER_TILE)], vb)
    pltpu.sync_copy(vb, degp_hbm.at[c].at[pl.ds(s * ROWS_PER_TILE,
                                                ROWS_PER_TILE)])


def _k1b_body(degp_ref, dinv_ref):
    dinv_ref[...] = lax.rsqrt(
        jnp.maximum(degp_ref[0] + degp_ref[1], 1e-6))[None, :]


def _k1b_call(degp):
    nblk = 10
    cb = NP // nblk
    return pl.pallas_call(
        _k1b_body,
        grid=(nblk,),
        in_specs=[pl.BlockSpec((NC, cb), lambda i: (0, i))],
        out_specs=pl.BlockSpec((1, cb), lambda i: (0, i)),
        out_shape=jax.ShapeDtypeStruct((1, NP), jnp.float32),
    )(degp)


PER_SC = NCHUNK // NC


@functools.partial(
    pl.kernel,
    out_type=jax.ShapeDtypeStruct((NC, NP, D), jnp.float32),
    mesh=_mesh(),
    scratch_types=[
        pltpu.VMEM_SHARED((NP, D), jnp.float32),
        pltpu.VMEM((3, CH), jnp.int32),
        pltpu.VMEM((3, CH), jnp.int32),
        pltpu.VMEM((3, CH), jnp.float32),
        pltpu.VMEM((2, CH), jnp.float32),
        pltpu.VMEM((2, CH), jnp.float32),
        pltpu.VMEM((CH,), jnp.float32),
        pltpu.VMEM((2, CH, D), jnp.float32),
        pltpu.SemaphoreType.DMA((3,)),
        pltpu.SemaphoreType.DMA((3,)),
    ],
)
def _k2_agg(src_hbm, dst_hbm, w_hbm, dinv_hbm, x_hbm, aggp_hbm,
            agg_sp, sidx, didx, wsl, nsrc, ndst, nbuf, xrows, semi, semg):
    c = lax.axis_index("c")
    s = lax.axis_index("s")

    for r in range(64):
        for k in range(D // L):
            xrows[0, r, pl.ds(k * L, L)] = jnp.zeros((L,), jnp.float32)

    def zstep(j, _):
        pltpu.sync_copy(xrows.at[0].at[pl.ds(0, 64), :],
                        agg_sp.at[pl.ds(s * ROWS_PER_TILE + j * 64, 64), :])
        return 0
    lax.fori_loop(0, ROWS_PER_TILE // 64, zstep, 0)
    plsc.subcore_barrier()

    nch = 78 + jnp.where(s < PER_SC - 78 * NS, 1, 0)

    def chunk_of(i):
        return c * PER_SC + s + NS * i

    def start_idx(i):
        sl = lax.rem(i, 3)
        ck = chunk_of(i)
        pltpu.async_copy(src_hbm.at[ck], sidx.at[sl], semi.at[sl])
        pltpu.async_copy(dst_hbm.at[ck], didx.at[sl], semi.at[sl])
        pltpu.async_copy(w_hbm.at[ck], wsl.at[sl], semi.at[sl])

    def wait_idx(i):
        sl = lax.rem(i, 3)
        ck = chunk_of(i)
        pltpu.make_async_copy(src_hbm.at[ck], sidx.at[sl], semi.at[sl]).wait()
        pltpu.make_async_copy(dst_hbm.at[ck], didx.at[sl], semi.at[sl]).wait()
        pltpu.make_async_copy(w_hbm.at[ck], wsl.at[sl], semi.at[sl]).wait()

    def start_gather(i):
        sl = lax.rem(i, 3)
        x2 = lax.rem(i, 2)
        pltpu.async_copy(dinv_hbm.at[sidx.at[sl]], nsrc.at[x2], semg.at[sl])
        pltpu.async_copy(dinv_hbm.at[didx.at[sl]], ndst.at[x2], semg.at[sl])
        pltpu.async_copy(x_hbm.at[sidx.at[sl]], xrows.at[x2], semg.at[sl])

    def wait_gather(i):
        sl = lax.rem(i, 3)
        x2 = lax.rem(i, 2)
        pltpu.make_async_copy(dinv_hbm.at[sidx.at[sl]], nsrc.at[x2],
                              semg.at[sl]).wait()
        pltpu.make_async_copy(dinv_hbm.at[didx.at[sl]], ndst.at[x2],
                              semg.at[sl]).wait()
        pltpu.make_async_copy(x_hbm.at[sidx.at[sl]], xrows.at[x2],
                              semg.at[sl]).wait()

    start_idx(0)

    @pl.when(nch > 1)
    def _():
        start_idx(1)
    wait_idx(0)
    start_gather(0)

    def step(i, _):
        sl = lax.rem(i, 3)

        @pl.when(i + 1 < nch)
        def _():
            wait_idx(i + 1)
            start_gather(i + 1)

        @pl.when(i + 2 < nch)
        def _():
            start_idx(i + 2)

        wait_gather(i)
        x2 = lax.rem(i, 2)
        for k in range(CH // L):
            nbuf[pl.ds(k * L, L)] = (nsrc[x2, pl.ds(k * L, L)]
                                     * ndst[x2, pl.ds(k * L, L)]
                                     * wsl[sl, pl.ds(k * L, L)])

        def scale(g, _):
            nv = nbuf[pl.ds(pl.multiple_of(g * L, L), L)]
            for j in range(L):
                e = g * L + j
                sc = nv[j]
                for k in range(D // L):
                    xrows[x2, e, pl.ds(k * L, L)] = (
                        xrows[x2, e, pl.ds(k * L, L)] * sc)
            return 0
        lax.fori_loop(0, CH // L, scale, 0)
        pltpu.sync_copy(xrows.at[x2], agg_sp.at[didx.at[sl]], add=True)
        return 0
    lax.fori_loop(0, nch, step, 0)
    plsc.subcore_barrier()

    def ostep(j, _):
        r = s * ROWS_PER_TILE + j * 64
        pltpu.sync_copy(agg_sp.at[pl.ds(r, 64), :],
                        xrows.at[0].at[pl.ds(0, 64), :])
        pltpu.sync_copy(xrows.at[0].at[pl.ds(0, 64), :],
                        aggp_hbm.at[c].at[pl.ds(r, 64), :])
        return 0
    lax.fori_loop(0, ROWS_PER_TILE // 64, ostep, 0)


def _k3_body(aggp_ref, wg_ref, bg_ref, w1a_ref, w1b_ref, b1_ref,
             a_ref, b_ref):
    agg = aggp_ref[0] + aggp_ref[1]
    emb = jnp.maximum(
        jnp.dot(agg, wg_ref[...], preferred_element_type=jnp.float32)
        + bg_ref[...], 0.0)
    a_ref[...] = (jnp.dot(emb, w1a_ref[...], preferred_element_type=jnp.float32)
                  + b1_ref[...])
    b_ref[...] = jnp.dot(emb, w1b_ref[...], preferred_element_type=jnp.float32)


def _k3_call(aggp, wg, bg, w1a, w1b, b1):
    nblk = 10
    rb = NP // nblk
    return pl.pallas_call(
        _k3_body,
        grid=(nblk,),
        in_specs=[
            pl.BlockSpec((NC, rb, D), lambda i: (0, i, 0)),
            pl.BlockSpec((D, H), lambda i: (0, 0)),
            pl.BlockSpec((1, H), lambda i: (0, 0)),
            pl.BlockSpec((H, H), lambda i: (0, 0)),
            pl.BlockSpec((H, H), lambda i: (0, 0)),
            pl.BlockSpec((1, H), lambda i: (0, 0)),
        ],
        out_specs=[
            pl.BlockSpec((rb, H), lambda i: (i, 0)),
            pl.BlockSpec((rb, H), lambda i: (i, 0)),
        ],
        out_shape=[
            jax.ShapeDtypeStruct((NP, H), jnp.float32),
            jax.ShapeDtypeStruct((NP, H), jnp.float32),
        ],
    )(aggp, wg, bg, w1a, w1b, b1)


@functools.partial(
    pl.kernel,
    out_type=jax.ShapeDtypeStruct((E2,), jnp.float32),
    mesh=_mesh(),
    scratch_types=[
        pltpu.VMEM((3, CH), jnp.int32),
        pltpu.VMEM((3, CH), jnp.int32),
        pltpu.VMEM((3, CH, H), jnp.float32),
        pltpu.VMEM((3, CH, H), jnp.float32),
        pltpu.VMEM((144,), jnp.float32),
        pltpu.VMEM((3, CH), jnp.float32),
        pltpu.SemaphoreType.DMA((3,)),
        pltpu.SemaphoreType.DMA((3,)),
        pltpu.SemaphoreType.DMA((3,)),
    ],
)
def _k4_decode(src_hbm, dst_hbm, a_hbm, b_hbm, par_hbm, out_hbm,
               sidx, didx, arows, brows, parv, obuf, semi, semg, semo):
    c = lax.axis_index("c")
    s = lax.axis_index("s")
    w = s * NC + c

    pltpu.sync_copy(par_hbm, parv)
    w2 = [parv[pl.ds(k * L, L)] for k in range(H // L)]
    b2 = parv[pl.ds(H, L)][0]
    i16 = lax.iota(jnp.int32, L)

    nch = 78 + jnp.where(w < NCHUNK - 78 * NW, 1, 0)

    def chunk_of(i):
        return w + NW * i

    def start_idx(i):
        sl = lax.rem(i, 3)
        ck = chunk_of(i)
        pltpu.async_copy(src_hbm.at[ck], sidx.at[sl], semi.at[sl])
        pltpu.async_copy(dst_hbm.at[ck], didx.at[sl], semi.at[sl])

    def wait_idx(i):
        sl = lax.rem(i, 3)
        ck = chunk_of(i)
        pltpu.make_async_copy(src_hbm.at[ck], sidx.at[sl], semi.at[sl]).wait()
        pltpu.make_async_copy(dst_hbm.at[ck], didx.at[sl], semi.at[sl]).wait()

    def start_gather(i):
        sl = lax.rem(i, 3)
        pltpu.async_copy(a_hbm.at[sidx.at[sl]], arows.at[sl], semg.at[sl])
        pltpu.async_copy(b_hbm.at[didx.at[sl]], brows.at[sl], semg.at[sl])

    def wait_gather(i):
        sl = lax.rem(i, 3)
        pltpu.make_async_copy(a_hbm.at[sidx.at[sl]], arows.at[sl],
                              semg.at[sl]).wait()
        pltpu.make_async_copy(b_hbm.at[didx.at[sl]], brows.at[sl],
                              semg.at[sl]).wait()

    def wait_out(i):
        sl = lax.rem(i, 3)
        base = chunk_of(i) * CH
        pltpu.make_async_copy(obuf.at[sl], out_hbm.at[pl.ds(base, CH)],
                              semo.at[sl]).wait()

    start_idx(0)

    @pl.when(nch > 1)
    def _():
        start_idx(1)
    wait_idx(0)
    start_gather(0)

    def step(i, _):
        sl = lax.rem(i, 3)

        @pl.when(i + 1 < nch)
        def _():
            wait_idx(i + 1)
            start_gather(i + 1)

        @pl.when(i + 2 < nch)
        def _():
            start_idx(i + 2)

        wait_gather(i)

        @pl.when(i >= 3)
        def _():
            wait_out(i - 3)

        def group(g, _):
            z = jnp.zeros((L,), jnp.float32)
            for j in range(L):
                e = g * L + j
                acc = jnp.zeros((L,), jnp.float32)
                for k in range(H // L):
                    t = jnp.maximum(
                        arows[sl, e, pl.ds(k * L, L)]
                        + brows[sl, e, pl.ds(k * L, L)], 0.0)
                    acc = acc + t * w2[k]
                sval = acc[0]
                for m in range(1, L):
                    sval = sval + acc[m]
                z = jnp.where(i16 == j, sval, z)
            sig = 1.0 / (1.0 + jnp.exp(-(z + b2)))
            obuf[sl, pl.ds(pl.multiple_of(g * L, L), L)] = sig
            return 0
        lax.fori_loop(0, CH // L, group, 0)
        base = chunk_of(i) * CH
        pltpu.async_copy(obuf.at[sl], out_hbm.at[pl.ds(base, CH)],
                         semo.at[sl])
        return 0
    lax.fori_loop(0, nch, step, 0)

    wait_out(nch - 3)
    wait_out(nch - 2)
    wait_out(nch - 1)


def kernel(x, edge_index, edge_weight, W_gcn, b_gcn, W1, b1, W2, b2):
    pad = E2 - E
    src = jnp.pad(edge_index[0], (0, pad)).reshape(NCHUNK, CH)
    dst = jnp.pad(edge_index[1], (0, pad)).reshape(NCHUNK, CH)
    wgt = jnp.pad(edge_weight, (0, pad)).reshape(NCHUNK, CH)
    degp = _k1_deg(dst.reshape(NG, G8, CH), wgt.reshape(NG, G8, CH))
    dinv = _k1b_call(degp).reshape(NP)
    aggp = _k2_agg(src, dst, wgt, dinv, x)
    a_mat, b_mat = _k3_call(
        aggp, W_gcn, b_gcn.reshape(1, H), W1[:H], W1[H:], b1.reshape(1, H))
    params = jnp.concatenate(
        [W2[:, 0], b2, jnp.zeros((15,), jnp.float32)])
    out = _k4_decode(src, dst, a_mat, b_mat, params)
    return out[:E]
